# baseline (speedup 1.0000x reference)
import functools

import jax
import jax.numpy as jnp
from jax import lax
from jax.experimental import pallas as pl
from jax.experimental.pallas import tpu as pltpu
from jax.experimental.pallas import tpu_sc as plsc

_NC = 2
_NS = 16
_NW = _NC * _NS
_CHUNK = 128


def _ring_for(feat):
    return (64, 3, 2) if feat >= 128 else (_CHUNK, 4, 1)


def _share_for(feat):
    return 0.5 if feat <= 16 else 0.77


def _layout(e, feat):
    chunk, nbuf, phases = _ring_for(feat)
    q = chunk * nbuf * phases
    per0 = int(_share_for(feat) * e / _NS)
    epc0 = -(-per0 // q) * q
    rem = max(e - _NS * epc0, 0)
    per1 = -(-rem // _NS)
    epc1 = max(-(-per1 // q) * q, q)
    return epc0, epc1


def _make_spmm(np_rows, feat, epc0, epc1):
    rows_per_tile = np_rows // _NS
    chunk, nbuf, phases = _ring_for(feat)
    ch0 = epc0 // chunk
    ng0 = ch0 // nbuf
    ng1 = (epc1 // chunk) // nbuf
    chp = ch0 // phases
    gpp = chp // nbuf
    mesh = plsc.VectorSubcoreMesh(
        core_axis_name="c", subcore_axis_name="s",
        num_cores=_NC, num_subcores=_NS)

    @functools.partial(
        pl.kernel,
        out_type=jax.ShapeDtypeStruct((_NC, np_rows, feat), jnp.float32),
        mesh=mesh,
        scratch_types=(
            [pltpu.VMEM((chp, chunk), jnp.int32),
             pltpu.VMEM((chp, chunk), jnp.int32)]
            + [pltpu.VMEM((chunk, feat), jnp.float32)] * nbuf
            + [pltpu.VMEM_SHARED((np_rows, feat), jnp.float32)]
            + [pltpu.SemaphoreType.DMA] * (2 * nbuf)
        ),
        compiler_params=pltpu.CompilerParams(use_tc_tiling_on_sc=False),
    )
    def spmm(table, gidx, sidx, zeros, out, gidx_v, sidx_v, *rest):
        bufs = rest[:nbuf]
        acc = rest[nbuf]
        gsems = rest[nbuf + 1:2 * nbuf + 1]
        ssems = rest[2 * nbuf + 1:]
        c = lax.axis_index("c")
        s = lax.axis_index("s")
        wid = c * _NS + s
        r0 = s * rows_per_tile
        ng_c = jnp.where(c == 0, ng0, ng1)
        with jax.named_scope("zero"):
            pltpu.sync_copy(zeros.at[pl.ds(r0, rows_per_tile)],
                            acc.at[pl.ds(r0, rows_per_tile)])
            plsc.subcore_barrier()

        for ph in range(phases):
            ngp = jnp.clip(ng_c - ph * gpp, 0, gpp)

            @pl.when(ngp > 0)
            def _(ph=ph, ngp=ngp):
                with jax.named_scope("stage"):
                    pltpu.sync_copy(gidx.at[wid, pl.ds(ph * chp, chp)],
                                    gidx_v)
                    pltpu.sync_copy(sidx.at[wid, pl.ds(ph * chp, chp)],
                                    sidx_v)
                for b in range(nbuf):
                    pltpu.async_copy(table.at[gidx_v.at[b]], bufs[b],
                                     gsems[b])

                def body(i, carry):
                    j0 = i * nbuf
                    for b in range(nbuf):
                        pltpu.make_async_copy(table.at[gidx_v.at[j0 + b]],
                                              bufs[b], gsems[b]).wait()
                        pltpu.async_copy(bufs[b], acc.at[sidx_v.at[j0 + b]],
                                         ssems[b], add=True)
                    for b in range(nbuf):
                        @pl.when(i + 1 < ngp)
                        def _(b=b, j0=j0):
                            pltpu.make_async_copy(
                                bufs[b], acc.at[sidx_v.at[j0 + b]],
                                ssems[b]).wait()
                            pltpu.async_copy(
                                table.at[gidx_v.at[j0 + nbuf + b]],
                                bufs[b], gsems[b])
                    return carry

                with jax.named_scope("ring"):
                    lax.fori_loop(0, ngp, body, 0)
                    for b in range(nbuf):
                        jlast = (ngp - 1) * nbuf + b
                        pltpu.make_async_copy(bufs[b],
                                              acc.at[sidx_v.at[jlast]],
                                              ssems[b]).wait()

        with jax.named_scope("cpout"):
            plsc.subcore_barrier()
            pltpu.sync_copy(acc.at[pl.ds(r0, rows_per_tile)],
                            out.at[c, pl.ds(r0, rows_per_tile)])

    return spmm


def _tc(fn, out_shapes, *args):
    return pl.pallas_call(fn, out_shape=out_shapes)(*args)


def _stage_enc(x, degraw, W_enc, b_enc, W1, np_rows, d):
    def f(x_ref, deg_ref, wenc_ref, benc_ref, w1_ref, dinv16_ref, y_ref, m1_ref):
        deg = deg_ref[0, :, 0:1] + deg_ref[1, :, 0:1] + 1.0
        dinv = lax.rsqrt(deg)
        dinv16_ref[...] = jnp.broadcast_to(dinv, dinv16_ref.shape)
        y = jnp.maximum(x_ref[...] @ wenc_ref[...] + benc_ref[...], 0.0)
        y_ref[...] = y
        wsum = w1_ref[0:d, :] + w1_ref[d:, :]
        m1_ref[...] = dinv * (y @ wsum)

    return _tc(f, [jax.ShapeDtypeStruct((np_rows, 16), jnp.float32),
                   jax.ShapeDtypeStruct((np_rows, d), jnp.float32),
                   jax.ShapeDtypeStruct((np_rows, 2 * d), jnp.float32)],
               x, degraw, W_enc, b_enc, W1)


def _stage_fwd1(s1, m1, dinv16, b1, W2, np_rows, d):
    def f(s1_ref, m1_ref, dinv_ref, b1_ref, w2_ref, o1_ref, m2_ref):
        dinv = dinv_ref[:, 0:1]
        z1 = dinv * (s1_ref[0] + s1_ref[1] + m1_ref[...]) + b1_ref[...]
        o1 = jnp.tanh(z1)
        o1_ref[...] = o1
        m2_ref[...] = dinv * (o1 @ w2_ref[...])

    return _tc(f, [jax.ShapeDtypeStruct((np_rows, 2 * d), jnp.float32),
                   jax.ShapeDtypeStruct((np_rows, d), jnp.float32)],
               s1, m1, dinv16, b1, W2)


def _stage_fwd2(s2, m2, dinv16, craw, b2, w3row, np_rows, d):
    def f(s2_ref, m2_ref, dinv_ref, craw_ref, b2_ref, w3_ref, g2_ref):
        dinv = dinv_ref[:, 0:1]
        z2 = dinv * (s2_ref[0] + s2_ref[1] + m2_ref[...]) + b2_ref[...]
        o2 = jnp.tanh(z2)
        c = dinv * (craw_ref[0, :, 0:1] + craw_ref[1, :, 0:1]) + dinv * dinv
        gz2 = (c * w3_ref[...]) * (1.0 - o2 * o2)
        g2_ref[...] = dinv * gz2

    return _tc(f, jax.ShapeDtypeStruct((np_rows, d), jnp.float32),
               s2, m2, dinv16, craw, b2, w3row)


def _stage_bwd1(t2, g2, dinv16, W2T, o1, np_rows, d):
    def f(t2_ref, g2_ref, dinv_ref, w2t_ref, o1_ref, g1_ref):
        dinv = dinv_ref[:, 0:1]
        atg2 = dinv * (t2_ref[0] + t2_ref[1] + g2_ref[...])
        go1 = atg2 @ w2t_ref[...]
        o1 = o1_ref[...]
        g1_ref[...] = dinv * (go1 * (1.0 - o1 * o1))

    return _tc(f, jax.ShapeDtypeStruct((np_rows, 2 * d), jnp.float32),
               t2, g2, dinv16, W2T, o1)


def _stage_update(t1, g1, dinv16, W1T, W1, Xc, Yc, np_rows, d):
    def f(t1_ref, g1_ref, dinv_ref, w1t_ref, w1_ref, x_ref, y_ref,
          xn_ref, yn_ref, m1_ref):
        dinv = dinv_ref[:, 0:1]
        g = (dinv * (t1_ref[0] + t1_ref[1] + g1_ref[...])) @ w1t_ref[...]
        xn = x_ref[...] + g[:, d:]
        yn = y_ref[...] - g[:, :d]
        xn_ref[...] = xn
        yn_ref[...] = yn
        m1_ref[...] = dinv * (xn @ w1_ref[0:d, :] + yn @ w1_ref[d:, :])

    return _tc(f, [jax.ShapeDtypeStruct((np_rows, d), jnp.float32),
                   jax.ShapeDtypeStruct((np_rows, d), jnp.float32),
                   jax.ShapeDtypeStruct((np_rows, 2 * d), jnp.float32)],
               t1, g1, dinv16, W1T, W1, Xc, Yc)


def _stage_dec(Xc, W_dec, b_dec, np_rows, nclass):
    def f(x_ref, wdec_ref, bdec_ref, out_ref):
        out_ref[...] = x_ref[...] @ wdec_ref[...] + bdec_ref[...]

    return _tc(f, jax.ShapeDtypeStruct((np_rows, nclass), jnp.float32),
               Xc, W_dec, b_dec)


def _pad_idx(idx, feat, fill):
    e = idx.shape[0]
    chunk, _, _ = _ring_for(feat)
    epc0, epc1 = _layout(e, feat)
    n0 = _NS * epc0
    total = _NS * (epc0 + epc1)
    flat = jnp.concatenate([idx, jnp.full((total - e,), fill, jnp.int32)])
    ch0, ch1 = epc0 // chunk, epc1 // chunk
    p0 = flat[:n0].reshape(_NS, ch0, chunk)
    p1 = flat[n0:].reshape(_NS, ch1, chunk)
    p1 = jnp.concatenate(
        [p1, jnp.full((_NS, ch0 - ch1, chunk), fill, jnp.int32)], axis=1)
    return jnp.concatenate([p0, p1], axis=0)


def kernel(x, edge_index, W_enc, b_enc, W1, b1, W2, b2, W3, b3, W_dec, b_dec):
    n = x.shape[0]
    e = edge_index.shape[1]
    d = W_enc.shape[1]
    nclass = W_dec.shape[1]
    nlayers = 2

    np_rows = ((n + _NS * 8 - 1) // (_NS * 8)) * (_NS * 8)
    trash = n

    src = edge_index[0].astype(jnp.int32)
    dst = edge_index[1].astype(jnp.int32)
    src_g16, src_s16 = _pad_idx(src, 16, 0), _pad_idx(src, 16, trash)
    dst_g16, dst_s16 = _pad_idx(dst, 16, 0), _pad_idx(dst, 16, trash)
    sg64, ss64 = _pad_idx(src, d, 0), _pad_idx(src, d, trash)
    dg64, ds64 = _pad_idx(dst, d, 0), _pad_idx(dst, d, trash)
    sg128, ss128 = _pad_idx(src, 2 * d, 0), _pad_idx(src, 2 * d, trash)
    dg128, ds128 = _pad_idx(dst, 2 * d, 0), _pad_idx(dst, 2 * d, trash)
    tot16 = src_g16.shape[0] * src_g16.shape[1] * src_g16.shape[2]
    zidx = jnp.tile(jnp.arange(128, dtype=jnp.int32),
                    tot16 // 128).reshape(src_g16.shape)

    zeros16 = jnp.zeros((np_rows, 16), jnp.float32)
    zeros64 = jnp.zeros((np_rows, d), jnp.float32)
    zeros128 = jnp.zeros((np_rows, 2 * d), jnp.float32)
    ones_tab = jnp.ones((128, 16), jnp.float32)

    x_p = jnp.concatenate(
        [x, jnp.zeros((np_rows - n, x.shape[1]), jnp.float32)])
    b_enc_r = b_enc.reshape(1, d)
    b1_r = b1.reshape(1, 2 * d)
    b2_r = b2.reshape(1, d)
    w3row = W3.reshape(1, d)
    b_dec_r = b_dec.reshape(1, nclass)
    W1T = W1.T
    W2T = W2.T

    sc16 = _make_spmm(np_rows, 16, *_layout(e, 16))
    sc64 = _make_spmm(np_rows, d, *_layout(e, d))
    sc128 = _make_spmm(np_rows, 2 * d, *_layout(e, 2 * d))

    degraw = sc16(ones_tab, zidx, dst_s16, zeros16)
    dinv16, Y, M1 = _stage_enc(x_p, degraw, W_enc, b_enc_r, W1, np_rows, d)
    craw = sc16(dinv16, dst_g16, src_s16, zeros16)

    X = Y
    for layer in range(nlayers):
        S1 = sc128(M1, sg128, ds128, zeros128)
        o1, M2 = _stage_fwd1(S1, M1, dinv16, b1_r, W2, np_rows, d)
        S2 = sc64(M2, sg64, ds64, zeros64)
        G2 = _stage_fwd2(S2, M2, dinv16, craw, b2_r, w3row, np_rows, d)
        T2 = sc64(G2, dg64, ss64, zeros64)
        G1 = _stage_bwd1(T2, G2, dinv16, W2T, o1, np_rows, d)
        T1 = sc128(G1, dg128, ss128, zeros128)
        X, Y, M1 = _stage_update(T1, G1, dinv16, W1T, W1, X, Y, np_rows, d)

    out = _stage_dec(X, W_dec, b_dec_r, np_rows, nclass)
    return out[:n]


# speedup vs baseline: 2.2599x; 2.2599x over previous
import functools

import jax
import jax.numpy as jnp
from jax import lax
from jax.experimental import pallas as pl
from jax.experimental.pallas import tpu as pltpu
from jax.experimental.pallas import tpu_sc as plsc

_NC = 2
_NS = 16
_NW = _NC * _NS
_CHUNK = 128


def _ring_for(feat):
    return (64, 3, 2) if feat >= 128 else (_CHUNK, 4, 1)


def _share_for(feat):
    return 0.5


def _layout(e, feat):
    chunk, nbuf, phases = _ring_for(feat)
    q = chunk * nbuf * phases
    per0 = int(_share_for(feat) * e / _NS)
    epc0 = -(-per0 // q) * q
    rem = max(e - _NS * epc0, 0)
    per1 = -(-rem // _NS)
    epc1 = max(-(-per1 // q) * q, q)
    return epc0, epc1


def _make_spmm(np_rows, feat, epc0, epc1):
    rows_per_tile = np_rows // _NS
    chunk, nbuf, phases = _ring_for(feat)
    ch0 = epc0 // chunk
    ng0 = ch0 // nbuf
    ng1 = (epc1 // chunk) // nbuf
    chp = ch0 // phases
    gpp = chp // nbuf
    mesh = plsc.VectorSubcoreMesh(
        core_axis_name="c", subcore_axis_name="s",
        num_cores=_NC, num_subcores=_NS)

    @functools.partial(
        pl.kernel,
        out_type=jax.ShapeDtypeStruct((_NC, np_rows, feat), jnp.float32),
        mesh=mesh,
        scratch_types=(
            [pltpu.VMEM((chp, chunk), jnp.int32),
             pltpu.VMEM((chp, chunk), jnp.int32)]
            + [pltpu.VMEM((chunk, feat), jnp.float32)] * nbuf
            + [pltpu.VMEM_SHARED((np_rows, feat), jnp.float32)]
            + [pltpu.SemaphoreType.DMA] * (2 * nbuf)
        ),
        compiler_params=pltpu.CompilerParams(use_tc_tiling_on_sc=False),
    )
    def spmm(table, gidx, sidx, zeros, out, gidx_v, sidx_v, *rest):
        bufs = rest[:nbuf]
        acc = rest[nbuf]
        gsems = rest[nbuf + 1:2 * nbuf + 1]
        ssems = rest[2 * nbuf + 1:]
        c = lax.axis_index("c")
        s = lax.axis_index("s")
        wid = c * _NS + s
        r0 = s * rows_per_tile
        ng_c = jnp.where(c == 0, ng0, ng1)
        with jax.named_scope("zero"):
            pltpu.sync_copy(zeros.at[pl.ds(r0, rows_per_tile)],
                            acc.at[pl.ds(r0, rows_per_tile)])
            plsc.subcore_barrier()

        for ph in range(phases):
            ngp = jnp.clip(ng_c - ph * gpp, 0, gpp)

            @pl.when(ngp > 0)
            def _(ph=ph, ngp=ngp):
                with jax.named_scope("stage"):
                    pltpu.sync_copy(gidx.at[wid, pl.ds(ph * chp, chp)],
                                    gidx_v)
                    pltpu.sync_copy(sidx.at[wid, pl.ds(ph * chp, chp)],
                                    sidx_v)
                for b in range(nbuf):
                    pltpu.async_copy(table.at[gidx_v.at[b]], bufs[b],
                                     gsems[b])

                def body(i, carry):
                    j0 = i * nbuf
                    for b in range(nbuf):
                        pltpu.make_async_copy(table.at[gidx_v.at[j0 + b]],
                                              bufs[b], gsems[b]).wait()
                        pltpu.async_copy(bufs[b], acc.at[sidx_v.at[j0 + b]],
                                         ssems[b], add=True)
                    for b in range(nbuf):
                        @pl.when(i + 1 < ngp)
                        def _(b=b, j0=j0):
                            pltpu.make_async_copy(
                                bufs[b], acc.at[sidx_v.at[j0 + b]],
                                ssems[b]).wait()
                            pltpu.async_copy(
                                table.at[gidx_v.at[j0 + nbuf + b]],
                                bufs[b], gsems[b])
                    return carry

                with jax.named_scope("ring"):
                    lax.fori_loop(0, ngp, body, 0)
                    for b in range(nbuf):
                        jlast = (ngp - 1) * nbuf + b
                        pltpu.make_async_copy(bufs[b],
                                              acc.at[sidx_v.at[jlast]],
                                              ssems[b]).wait()

        with jax.named_scope("cpout"):
            plsc.subcore_barrier()
            pltpu.sync_copy(acc.at[pl.ds(r0, rows_per_tile)],
                            out.at[c, pl.ds(r0, rows_per_tile)])

    return spmm


def _tc(fn, out_shapes, *args):
    return pl.pallas_call(fn, out_shape=out_shapes)(*args)


def _stage_enc(x, degraw, W_enc, b_enc, W1, np_rows, d, n):
    def f(x_ref, deg_ref, wenc_ref, benc_ref, w1_ref, dinv16_ref, y_ref, m1_ref):
        deg = deg_ref[0, :, 0:1] + deg_ref[1, :, 0:1] + 1.0
        row = lax.broadcasted_iota(jnp.int32, (np_rows, 1), 0)
        dinv = jnp.where(row < n, lax.rsqrt(deg), 0.0)
        dinv16_ref[...] = jnp.broadcast_to(dinv, dinv16_ref.shape)
        y = jnp.maximum(x_ref[...] @ wenc_ref[...] + benc_ref[...], 0.0)
        y_ref[...] = y
        wsum = w1_ref[0:d, :] + w1_ref[d:, :]
        m1_ref[...] = dinv * (y @ wsum)

    return _tc(f, [jax.ShapeDtypeStruct((np_rows, 16), jnp.float32),
                   jax.ShapeDtypeStruct((np_rows, d), jnp.float32),
                   jax.ShapeDtypeStruct((np_rows, 2 * d), jnp.float32)],
               x, degraw, W_enc, b_enc, W1)


def _stage_fwd1(s1, m1, dinv16, b1, W2, np_rows, d):
    def f(s1_ref, m1_ref, dinv_ref, b1_ref, w2_ref, o1_ref, m2_ref):
        dinv = dinv_ref[:, 0:1]
        z1 = dinv * (s1_ref[0] + s1_ref[1] + m1_ref[...]) + b1_ref[...]
        o1 = jnp.tanh(z1)
        o1_ref[...] = o1
        m2_ref[...] = dinv * (o1 @ w2_ref[...])

    return _tc(f, [jax.ShapeDtypeStruct((np_rows, 2 * d), jnp.float32),
                   jax.ShapeDtypeStruct((np_rows, d), jnp.float32)],
               s1, m1, dinv16, b1, W2)


def _stage_fwd2(s2, m2, dinv16, craw, b2, w3row, np_rows, d):
    def f(s2_ref, m2_ref, dinv_ref, craw_ref, b2_ref, w3_ref, g2_ref):
        dinv = dinv_ref[:, 0:1]
        z2 = dinv * (s2_ref[0] + s2_ref[1] + m2_ref[...]) + b2_ref[...]
        o2 = jnp.tanh(z2)
        c = dinv * (craw_ref[0, :, 0:1] + craw_ref[1, :, 0:1]) + dinv * dinv
        gz2 = (c * w3_ref[...]) * (1.0 - o2 * o2)
        g2_ref[...] = dinv * gz2

    return _tc(f, jax.ShapeDtypeStruct((np_rows, d), jnp.float32),
               s2, m2, dinv16, craw, b2, w3row)


def _stage_bwd1(t2, g2, dinv16, W2T, o1, np_rows, d):
    def f(t2_ref, g2_ref, dinv_ref, w2t_ref, o1_ref, g1_ref):
        dinv = dinv_ref[:, 0:1]
        atg2 = dinv * (t2_ref[0] + t2_ref[1] + g2_ref[...])
        go1 = atg2 @ w2t_ref[...]
        o1 = o1_ref[...]
        g1_ref[...] = dinv * (go1 * (1.0 - o1 * o1))

    return _tc(f, jax.ShapeDtypeStruct((np_rows, 2 * d), jnp.float32),
               t2, g2, dinv16, W2T, o1)


def _stage_update(t1, g1, dinv16, W1T, W1, Xc, Yc, np_rows, d):
    def f(t1_ref, g1_ref, dinv_ref, w1t_ref, w1_ref, x_ref, y_ref,
          xn_ref, yn_ref, m1_ref):
        dinv = dinv_ref[:, 0:1]
        g = (dinv * (t1_ref[0] + t1_ref[1] + g1_ref[...])) @ w1t_ref[...]
        xn = x_ref[...] + g[:, d:]
        yn = y_ref[...] - g[:, :d]
        xn_ref[...] = xn
        yn_ref[...] = yn
        m1_ref[...] = dinv * (xn @ w1_ref[0:d, :] + yn @ w1_ref[d:, :])

    return _tc(f, [jax.ShapeDtypeStruct((np_rows, d), jnp.float32),
                   jax.ShapeDtypeStruct((np_rows, d), jnp.float32),
                   jax.ShapeDtypeStruct((np_rows, 2 * d), jnp.float32)],
               t1, g1, dinv16, W1T, W1, Xc, Yc)


def _stage_dec(Xc, W_dec, b_dec, np_rows, nclass):
    def f(x_ref, wdec_ref, bdec_ref, out_ref):
        out_ref[...] = x_ref[...] @ wdec_ref[...] + bdec_ref[...]

    return _tc(f, jax.ShapeDtypeStruct((np_rows, nclass), jnp.float32),
               Xc, W_dec, b_dec)


def _pad_idx(idx, feat, fillvec):
    e = idx.shape[0]
    chunk, _, _ = _ring_for(feat)
    epc0, epc1 = _layout(e, feat)
    n0 = _NS * epc0
    total = _NS * (epc0 + epc1)
    npad = total - e
    fill = jnp.tile(fillvec, -(-npad // fillvec.shape[0]))[:npad]
    flat = jnp.concatenate([idx, fill])
    ch0, ch1 = epc0 // chunk, epc1 // chunk
    p0 = flat[:n0].reshape(_NS, ch0, chunk)
    p1 = flat[n0:].reshape(_NS, ch1, chunk)
    if ch0 > ch1:
        extra = jnp.tile(fillvec, -(-(_NS * (ch0 - ch1) * chunk)
                                    // fillvec.shape[0]))
        extra = extra[:_NS * (ch0 - ch1) * chunk].reshape(
            _NS, ch0 - ch1, chunk)
        p1 = jnp.concatenate([p1, extra], axis=1)
    return jnp.concatenate([p0, p1], axis=0)


def kernel(x, edge_index, W_enc, b_enc, W1, b1, W2, b2, W3, b3, W_dec, b_dec):
    n = x.shape[0]
    e = edge_index.shape[1]
    d = W_enc.shape[1]
    nclass = W_dec.shape[1]
    nlayers = 2

    np_rows = -(-(n + 128) // _NS) * _NS

    src = edge_index[0].astype(jnp.int32)
    dst = edge_index[1].astype(jnp.int32)
    cyc = jnp.arange(128, dtype=jnp.int32)
    gfill = n + cyc
    sfill = (cyc * 79) % n
    src_g16, src_s16 = _pad_idx(src, 16, gfill), _pad_idx(src, 16, sfill)
    dst_g16, dst_s16 = _pad_idx(dst, 16, gfill), _pad_idx(dst, 16, sfill)
    sg64, ss64 = _pad_idx(src, d, gfill), _pad_idx(src, d, sfill)
    dg64, ds64 = _pad_idx(dst, d, gfill), _pad_idx(dst, d, sfill)
    sg128, ss128 = _pad_idx(src, 2 * d, gfill), _pad_idx(src, 2 * d, sfill)
    dg128, ds128 = _pad_idx(dst, 2 * d, gfill), _pad_idx(dst, 2 * d, sfill)
    zidx = _pad_idx(jnp.arange(e, dtype=jnp.int32) % 128, 16, 128 + cyc)

    zeros16 = jnp.zeros((np_rows, 16), jnp.float32)
    zeros64 = jnp.zeros((np_rows, d), jnp.float32)
    zeros128 = jnp.zeros((np_rows, 2 * d), jnp.float32)
    ones_tab = jnp.concatenate([jnp.ones((128, 16), jnp.float32),
                                jnp.zeros((128, 16), jnp.float32)])

    x_p = jnp.concatenate(
        [x, jnp.zeros((np_rows - n, x.shape[1]), jnp.float32)])
    b_enc_r = b_enc.reshape(1, d)
    b1_r = b1.reshape(1, 2 * d)
    b2_r = b2.reshape(1, d)
    w3row = W3.reshape(1, d)
    b_dec_r = b_dec.reshape(1, nclass)
    W1T = W1.T
    W2T = W2.T

    sc16 = _make_spmm(np_rows, 16, *_layout(e, 16))
    sc64 = _make_spmm(np_rows, d, *_layout(e, d))
    sc128 = _make_spmm(np_rows, 2 * d, *_layout(e, 2 * d))

    degraw = sc16(ones_tab, zidx, dst_s16, zeros16)
    dinv16, Y, M1 = _stage_enc(x_p, degraw, W_enc, b_enc_r, W1, np_rows, d, n)
    craw = sc16(dinv16, dst_g16, src_s16, zeros16)

    X = Y
    for layer in range(nlayers):
        S1 = sc128(M1, sg128, ds128, zeros128)
        o1, M2 = _stage_fwd1(S1, M1, dinv16, b1_r, W2, np_rows, d)
        S2 = sc64(M2, sg64, ds64, zeros64)
        G2 = _stage_fwd2(S2, M2, dinv16, craw, b2_r, w3row, np_rows, d)
        T2 = sc64(G2, dg64, ss64, zeros64)
        G1 = _stage_bwd1(T2, G2, dinv16, W2T, o1, np_rows, d)
        T1 = sc128(G1, dg128, ss128, zeros128)
        X, Y, M1 = _stage_update(T1, G1, dinv16, W1T, W1, X, Y, np_rows, d)

    out = _stage_dec(X, W_dec, b_dec_r, np_rows, nclass)
    return out[:n]


# speedup vs baseline: 2.4132x; 1.0679x over previous
import functools

import jax
import jax.numpy as jnp
from jax import lax
from jax.experimental import pallas as pl
from jax.experimental.pallas import tpu as pltpu
from jax.experimental.pallas import tpu_sc as plsc

_NC = 2
_NS = 16
_NW = _NC * _NS
_CHUNK = 128


def _ring_for(feat):
    if feat >= 128:
        return (64, 3, 2)
    if feat <= 16:
        return (_CHUNK, 8, 1)
    return (_CHUNK, 4, 1)


def _share_for(feat):
    return 0.5


def _layout(e, feat):
    chunk, nbuf, phases = _ring_for(feat)
    q = chunk * nbuf * phases
    per0 = int(_share_for(feat) * e / _NS)
    epc0 = -(-per0 // q) * q
    rem = max(e - _NS * epc0, 0)
    per1 = -(-rem // _NS)
    epc1 = max(-(-per1 // q) * q, q)
    return epc0, epc1


def _make_spmm(np_rows, feat, epc0, epc1):
    rows_per_tile = np_rows // _NS
    chunk, nbuf, phases = _ring_for(feat)
    ch0 = epc0 // chunk
    ng0 = ch0 // nbuf
    ng1 = (epc1 // chunk) // nbuf
    chp = ch0 // phases
    gpp = chp // nbuf
    mesh = plsc.VectorSubcoreMesh(
        core_axis_name="c", subcore_axis_name="s",
        num_cores=_NC, num_subcores=_NS)

    @functools.partial(
        pl.kernel,
        out_type=jax.ShapeDtypeStruct((_NC, np_rows, feat), jnp.float32),
        mesh=mesh,
        scratch_types=(
            [pltpu.VMEM((chp, chunk), jnp.int32),
             pltpu.VMEM((chp, chunk), jnp.int32)]
            + [pltpu.VMEM((chunk, feat), jnp.float32)] * nbuf
            + [pltpu.VMEM_SHARED((np_rows, feat), jnp.float32)]
            + [pltpu.SemaphoreType.DMA] * (2 * nbuf)
        ),
        compiler_params=pltpu.CompilerParams(use_tc_tiling_on_sc=False),
    )
    def spmm(table, gidx, sidx, zeros, out, gidx_v, sidx_v, *rest):
        bufs = rest[:nbuf]
        acc = rest[nbuf]
        gsems = rest[nbuf + 1:2 * nbuf + 1]
        ssems = rest[2 * nbuf + 1:]
        c = lax.axis_index("c")
        s = lax.axis_index("s")
        wid = c * _NS + s
        r0 = s * rows_per_tile
        ng_c = jnp.where(c == 0, ng0, ng1)
        with jax.named_scope("zero"):
            pltpu.sync_copy(zeros.at[pl.ds(r0, rows_per_tile)],
                            acc.at[pl.ds(r0, rows_per_tile)])
            plsc.subcore_barrier()

        for ph in range(phases):
            ngp = jnp.clip(ng_c - ph * gpp, 0, gpp)

            @pl.when(ngp > 0)
            def _(ph=ph, ngp=ngp):
                with jax.named_scope("stage"):
                    pltpu.sync_copy(gidx.at[wid, pl.ds(ph * chp, chp)],
                                    gidx_v)
                    pltpu.sync_copy(sidx.at[wid, pl.ds(ph * chp, chp)],
                                    sidx_v)
                for b in range(nbuf):
                    pltpu.async_copy(table.at[gidx_v.at[b]], bufs[b],
                                     gsems[b])

                def body(i, carry):
                    j0 = i * nbuf
                    for b in range(nbuf):
                        pltpu.make_async_copy(table.at[gidx_v.at[j0 + b]],
                                              bufs[b], gsems[b]).wait()
                        pltpu.async_copy(bufs[b], acc.at[sidx_v.at[j0 + b]],
                                         ssems[b], add=True)
                    for b in range(nbuf):
                        @pl.when(i + 1 < ngp)
                        def _(b=b, j0=j0):
                            pltpu.make_async_copy(
                                bufs[b], acc.at[sidx_v.at[j0 + b]],
                                ssems[b]).wait()
                            pltpu.async_copy(
                                table.at[gidx_v.at[j0 + nbuf + b]],
                                bufs[b], gsems[b])
                    return carry

                with jax.named_scope("ring"):
                    lax.fori_loop(0, ngp, body, 0)
                    for b in range(nbuf):
                        jlast = (ngp - 1) * nbuf + b
                        pltpu.make_async_copy(bufs[b],
                                              acc.at[sidx_v.at[jlast]],
                                              ssems[b]).wait()

        with jax.named_scope("cpout"):
            plsc.subcore_barrier()
            pltpu.sync_copy(acc.at[pl.ds(r0, rows_per_tile)],
                            out.at[c, pl.ds(r0, rows_per_tile)])

    return spmm


def _tc(fn, out_shapes, *args):
    return pl.pallas_call(fn, out_shape=out_shapes)(*args)


def _stage_enc(x, degraw, W_enc, b_enc, W1, np_rows, d, n):
    def f(x_ref, deg_ref, wenc_ref, benc_ref, w1_ref, dinv16_ref, y_ref, m1_ref):
        deg = deg_ref[0, :, 0:1] + deg_ref[1, :, 0:1] + 1.0
        row = lax.broadcasted_iota(jnp.int32, (np_rows, 1), 0)
        dinv = jnp.where(row < n, lax.rsqrt(deg), 0.0)
        dinv16_ref[...] = jnp.broadcast_to(dinv, dinv16_ref.shape)
        y = jnp.maximum(x_ref[...] @ wenc_ref[...] + benc_ref[...], 0.0)
        y_ref[...] = y
        wsum = w1_ref[0:d, :] + w1_ref[d:, :]
        m1_ref[...] = dinv * (y @ wsum)

    return _tc(f, [jax.ShapeDtypeStruct((np_rows, 16), jnp.float32),
                   jax.ShapeDtypeStruct((np_rows, d), jnp.float32),
                   jax.ShapeDtypeStruct((np_rows, 2 * d), jnp.float32)],
               x, degraw, W_enc, b_enc, W1)


def _stage_fwd1(s1, m1, dinv16, b1, W2, np_rows, d):
    def f(s1_ref, m1_ref, dinv_ref, b1_ref, w2_ref, o1_ref, m2_ref):
        dinv = dinv_ref[:, 0:1]
        z1 = dinv * (s1_ref[0] + s1_ref[1] + m1_ref[...]) + b1_ref[...]
        o1 = jnp.tanh(z1)
        o1_ref[...] = o1
        m2_ref[...] = dinv * (o1 @ w2_ref[...])

    return _tc(f, [jax.ShapeDtypeStruct((np_rows, 2 * d), jnp.float32),
                   jax.ShapeDtypeStruct((np_rows, d), jnp.float32)],
               s1, m1, dinv16, b1, W2)


def _stage_fwd2(s2, m2, dinv16, craw, b2, w3row, np_rows, d):
    def f(s2_ref, m2_ref, dinv_ref, craw_ref, b2_ref, w3_ref, g2_ref):
        dinv = dinv_ref[:, 0:1]
        z2 = dinv * (s2_ref[0] + s2_ref[1] + m2_ref[...]) + b2_ref[...]
        o2 = jnp.tanh(z2)
        c = dinv * (craw_ref[0, :, 0:1] + craw_ref[1, :, 0:1]) + dinv * dinv
        gz2 = (c * w3_ref[...]) * (1.0 - o2 * o2)
        g2_ref[...] = dinv * gz2

    return _tc(f, jax.ShapeDtypeStruct((np_rows, d), jnp.float32),
               s2, m2, dinv16, craw, b2, w3row)


def _stage_bwd1(t2, g2, dinv16, W2T, o1, np_rows, d):
    def f(t2_ref, g2_ref, dinv_ref, w2t_ref, o1_ref, g1_ref):
        dinv = dinv_ref[:, 0:1]
        atg2 = dinv * (t2_ref[0] + t2_ref[1] + g2_ref[...])
        go1 = atg2 @ w2t_ref[...]
        o1 = o1_ref[...]
        g1_ref[...] = dinv * (go1 * (1.0 - o1 * o1))

    return _tc(f, jax.ShapeDtypeStruct((np_rows, 2 * d), jnp.float32),
               t2, g2, dinv16, W2T, o1)


def _stage_update(t1, g1, dinv16, W1T, W1, Xc, Yc, np_rows, d):
    def f(t1_ref, g1_ref, dinv_ref, w1t_ref, w1_ref, x_ref, y_ref,
          xn_ref, yn_ref, m1_ref):
        dinv = dinv_ref[:, 0:1]
        g = (dinv * (t1_ref[0] + t1_ref[1] + g1_ref[...])) @ w1t_ref[...]
        xn = x_ref[...] + g[:, d:]
        yn = y_ref[...] - g[:, :d]
        xn_ref[...] = xn
        yn_ref[...] = yn
        m1_ref[...] = dinv * (xn @ w1_ref[0:d, :] + yn @ w1_ref[d:, :])

    return _tc(f, [jax.ShapeDtypeStruct((np_rows, d), jnp.float32),
                   jax.ShapeDtypeStruct((np_rows, d), jnp.float32),
                   jax.ShapeDtypeStruct((np_rows, 2 * d), jnp.float32)],
               t1, g1, dinv16, W1T, W1, Xc, Yc)


def _stage_dec(Xc, W_dec, b_dec, np_rows, nclass):
    def f(x_ref, wdec_ref, bdec_ref, out_ref):
        out_ref[...] = x_ref[...] @ wdec_ref[...] + bdec_ref[...]

    return _tc(f, jax.ShapeDtypeStruct((np_rows, nclass), jnp.float32),
               Xc, W_dec, b_dec)


def _pad_idx(idx, feat, fillvec):
    e = idx.shape[0]
    chunk, _, _ = _ring_for(feat)
    epc0, epc1 = _layout(e, feat)
    n0 = _NS * epc0
    total = _NS * (epc0 + epc1)
    npad = total - e
    fill = jnp.tile(fillvec, -(-npad // fillvec.shape[0]))[:npad]
    flat = jnp.concatenate([idx, fill])
    ch0, ch1 = epc0 // chunk, epc1 // chunk
    p0 = flat[:n0].reshape(_NS, ch0, chunk)
    p1 = flat[n0:].reshape(_NS, ch1, chunk)
    if ch0 > ch1:
        extra = jnp.tile(fillvec, -(-(_NS * (ch0 - ch1) * chunk)
                                    // fillvec.shape[0]))
        extra = extra[:_NS * (ch0 - ch1) * chunk].reshape(
            _NS, ch0 - ch1, chunk)
        p1 = jnp.concatenate([p1, extra], axis=1)
    return jnp.concatenate([p0, p1], axis=0)


def kernel(x, edge_index, W_enc, b_enc, W1, b1, W2, b2, W3, b3, W_dec, b_dec):
    n = x.shape[0]
    e = edge_index.shape[1]
    d = W_enc.shape[1]
    nclass = W_dec.shape[1]
    nlayers = 2

    np_rows = -(-(n + 128) // _NS) * _NS

    src = edge_index[0].astype(jnp.int32)
    dst = edge_index[1].astype(jnp.int32)
    cyc = jnp.arange(128, dtype=jnp.int32)
    gfill = n + cyc
    sfill = (cyc * 79) % n
    src_g16, src_s16 = _pad_idx(src, 16, gfill), _pad_idx(src, 16, sfill)
    dst_g16, dst_s16 = _pad_idx(dst, 16, gfill), _pad_idx(dst, 16, sfill)
    sg64, ss64 = _pad_idx(src, d, gfill), _pad_idx(src, d, sfill)
    dg64, ds64 = _pad_idx(dst, d, gfill), _pad_idx(dst, d, sfill)
    sg128, ss128 = _pad_idx(src, 2 * d, gfill), _pad_idx(src, 2 * d, sfill)
    dg128, ds128 = _pad_idx(dst, 2 * d, gfill), _pad_idx(dst, 2 * d, sfill)
    zidx = _pad_idx(jnp.arange(e, dtype=jnp.int32) % 2048, 16, 2048 + cyc)

    zeros16 = jnp.zeros((np_rows, 16), jnp.float32)
    zeros64 = jnp.zeros((np_rows, d), jnp.float32)
    zeros128 = jnp.zeros((np_rows, 2 * d), jnp.float32)
    ones_tab = jnp.concatenate([jnp.ones((2048, 16), jnp.float32),
                                jnp.zeros((128, 16), jnp.float32)])

    x_p = jnp.concatenate(
        [x, jnp.zeros((np_rows - n, x.shape[1]), jnp.float32)])
    b_enc_r = b_enc.reshape(1, d)
    b1_r = b1.reshape(1, 2 * d)
    b2_r = b2.reshape(1, d)
    w3row = W3.reshape(1, d)
    b_dec_r = b_dec.reshape(1, nclass)
    W1T = W1.T
    W2T = W2.T

    sc16 = _make_spmm(np_rows, 16, *_layout(e, 16))
    sc64 = _make_spmm(np_rows, d, *_layout(e, d))
    sc128 = _make_spmm(np_rows, 2 * d, *_layout(e, 2 * d))

    degraw = sc16(ones_tab, zidx, dst_s16, zeros16)
    dinv16, Y, M1 = _stage_enc(x_p, degraw, W_enc, b_enc_r, W1, np_rows, d, n)
    craw = sc16(dinv16, dst_g16, src_s16, zeros16)

    X = Y
    for layer in range(nlayers):
        S1 = sc128(M1, sg128, ds128, zeros128)
        o1, M2 = _stage_fwd1(S1, M1, dinv16, b1_r, W2, np_rows, d)
        S2 = sc64(M2, sg64, ds64, zeros64)
        G2 = _stage_fwd2(S2, M2, dinv16, craw, b2_r, w3row, np_rows, d)
        T2 = sc64(G2, dg64, ss64, zeros64)
        G1 = _stage_bwd1(T2, G2, dinv16, W2T, o1, np_rows, d)
        T1 = sc128(G1, dg128, ss128, zeros128)
        X, Y, M1 = _stage_update(T1, G1, dinv16, W1T, W1, X, Y, np_rows, d)

    out = _stage_dec(X, W_dec, b_dec_r, np_rows, nclass)
    return out[:n]


# speedup vs baseline: 2.4623x; 1.0203x over previous
import functools

import jax
import jax.numpy as jnp
from jax import lax
from jax.experimental import pallas as pl
from jax.experimental.pallas import tpu as pltpu
from jax.experimental.pallas import tpu_sc as plsc

_NC = 2
_NS = 16
_NW = _NC * _NS
_CHUNK = 128


def _ring_for(feat):
    return (64, 3, 2) if feat >= 128 else (_CHUNK, 4, 1)


def _share_for(feat):
    return 0.5


def _layout(e, feat):
    chunk, nbuf, phases = _ring_for(feat)
    q = chunk * nbuf * phases
    per0 = int(_share_for(feat) * e / _NS)
    epc0 = -(-per0 // q) * q
    rem = max(e - _NS * epc0, 0)
    per1 = -(-rem // _NS)
    epc1 = max(-(-per1 // q) * q, q)
    return epc0, epc1


def _make_spmm(np_rows, feat, epc0, epc1):
    rows_per_tile = np_rows // _NS
    chunk, nbuf, phases = _ring_for(feat)
    ch0 = epc0 // chunk
    ng0 = ch0 // nbuf
    ng1 = (epc1 // chunk) // nbuf
    chp = ch0 // phases
    gpp = chp // nbuf
    mesh = plsc.VectorSubcoreMesh(
        core_axis_name="c", subcore_axis_name="s",
        num_cores=_NC, num_subcores=_NS)

    @functools.partial(
        pl.kernel,
        out_type=jax.ShapeDtypeStruct((_NC, np_rows, feat), jnp.float32),
        mesh=mesh,
        scratch_types=(
            [pltpu.VMEM((chp, chunk), jnp.int32),
             pltpu.VMEM((chp, chunk), jnp.int32)]
            + [pltpu.VMEM((chunk, feat), jnp.float32)] * nbuf
            + [pltpu.VMEM_SHARED((np_rows, feat), jnp.float32)]
            + [pltpu.SemaphoreType.DMA] * (2 * nbuf)
        ),
        compiler_params=pltpu.CompilerParams(use_tc_tiling_on_sc=False),
    )
    def spmm(table, gidx, sidx, out, gidx_v, sidx_v, *rest):
        bufs = rest[:nbuf]
        acc = rest[nbuf]
        gsems = rest[nbuf + 1:2 * nbuf + 1]
        ssems = rest[2 * nbuf + 1:]
        c = lax.axis_index("c")
        s = lax.axis_index("s")
        wid = c * _NS + s
        r0 = s * rows_per_tile
        ng_c = jnp.where(c == 0, ng0, ng1)
        with jax.named_scope("zero"):
            zv = jnp.zeros((16,), jnp.float32)

            def zfill(i, carry):
                for kk in range(feat // 16):
                    bufs[0][i, pl.ds(kk * 16, 16)] = zv
                return carry

            lax.fori_loop(0, chunk, zfill, 0)
            nfull = rows_per_tile // chunk
            rem = rows_per_tile % chunk
            for k in range(nfull):
                pltpu.sync_copy(bufs[0],
                                acc.at[pl.ds(r0 + k * chunk, chunk)])
            if rem:
                pltpu.sync_copy(bufs[0].at[pl.ds(0, rem)],
                                acc.at[pl.ds(r0 + nfull * chunk, rem)])
            plsc.subcore_barrier()

        for ph in range(phases):
            ngp = jnp.clip(ng_c - ph * gpp, 0, gpp)

            @pl.when(ngp > 0)
            def _(ph=ph, ngp=ngp):
                with jax.named_scope("stage"):
                    pltpu.sync_copy(gidx.at[wid, pl.ds(ph * chp, chp)],
                                    gidx_v)
                    pltpu.sync_copy(sidx.at[wid, pl.ds(ph * chp, chp)],
                                    sidx_v)
                for b in range(nbuf):
                    pltpu.async_copy(table.at[gidx_v.at[b]], bufs[b],
                                     gsems[b])

                def body(i, carry):
                    j0 = i * nbuf
                    for b in range(nbuf):
                        pltpu.make_async_copy(table.at[gidx_v.at[j0 + b]],
                                              bufs[b], gsems[b]).wait()
                        pltpu.async_copy(bufs[b], acc.at[sidx_v.at[j0 + b]],
                                         ssems[b], add=True)
                    for b in range(nbuf):
                        @pl.when(i + 1 < ngp)
                        def _(b=b, j0=j0):
                            pltpu.make_async_copy(
                                bufs[b], acc.at[sidx_v.at[j0 + b]],
                                ssems[b]).wait()
                            pltpu.async_copy(
                                table.at[gidx_v.at[j0 + nbuf + b]],
                                bufs[b], gsems[b])
                    return carry

                with jax.named_scope("ring"):
                    lax.fori_loop(0, ngp, body, 0)
                    for b in range(nbuf):
                        jlast = (ngp - 1) * nbuf + b
                        pltpu.make_async_copy(bufs[b],
                                              acc.at[sidx_v.at[jlast]],
                                              ssems[b]).wait()

        with jax.named_scope("cpout"):
            plsc.subcore_barrier()
            pltpu.sync_copy(acc.at[pl.ds(r0, rows_per_tile)],
                            out.at[c, pl.ds(r0, rows_per_tile)])

    return spmm


def _tc(fn, out_shapes, *args):
    return pl.pallas_call(fn, out_shape=out_shapes)(*args)


def _stage_enc(x, degraw, W_enc, b_enc, W1, np_rows, d, n):
    def f(x_ref, deg_ref, wenc_ref, benc_ref, w1_ref, dinv16_ref, y_ref, m1_ref):
        deg = deg_ref[0, :, 0:1] + deg_ref[1, :, 0:1] + 1.0
        row = lax.broadcasted_iota(jnp.int32, (np_rows, 1), 0)
        dinv = jnp.where(row < n, lax.rsqrt(deg), 0.0)
        dinv16_ref[...] = jnp.broadcast_to(dinv, dinv16_ref.shape)
        y = jnp.maximum(x_ref[...] @ wenc_ref[...] + benc_ref[...], 0.0)
        y_ref[...] = y
        wsum = w1_ref[0:d, :] + w1_ref[d:, :]
        m1_ref[...] = dinv * (y @ wsum)

    return _tc(f, [jax.ShapeDtypeStruct((np_rows, 16), jnp.float32),
                   jax.ShapeDtypeStruct((np_rows, d), jnp.float32),
                   jax.ShapeDtypeStruct((np_rows, 2 * d), jnp.float32)],
               x, degraw, W_enc, b_enc, W1)


def _stage_fwd1(s1, m1, dinv16, b1, W2, np_rows, d):
    def f(s1_ref, m1_ref, dinv_ref, b1_ref, w2_ref, o1_ref, m2_ref):
        dinv = dinv_ref[:, 0:1]
        z1 = dinv * (s1_ref[0] + s1_ref[1] + m1_ref[...]) + b1_ref[...]
        o1 = jnp.tanh(z1)
        o1_ref[...] = o1
        m2_ref[...] = dinv * (o1 @ w2_ref[...])

    return _tc(f, [jax.ShapeDtypeStruct((np_rows, 2 * d), jnp.float32),
                   jax.ShapeDtypeStruct((np_rows, d), jnp.float32)],
               s1, m1, dinv16, b1, W2)


def _stage_fwd2(s2, m2, dinv16, craw, b2, w3row, np_rows, d):
    def f(s2_ref, m2_ref, dinv_ref, craw_ref, b2_ref, w3_ref, g2_ref):
        dinv = dinv_ref[:, 0:1]
        z2 = dinv * (s2_ref[0] + s2_ref[1] + m2_ref[...]) + b2_ref[...]
        o2 = jnp.tanh(z2)
        c = dinv * (craw_ref[0, :, 0:1] + craw_ref[1, :, 0:1]) + dinv * dinv
        gz2 = (c * w3_ref[...]) * (1.0 - o2 * o2)
        g2_ref[...] = dinv * gz2

    return _tc(f, jax.ShapeDtypeStruct((np_rows, d), jnp.float32),
               s2, m2, dinv16, craw, b2, w3row)


def _stage_bwd1(t2, g2, dinv16, W2T, o1, np_rows, d):
    def f(t2_ref, g2_ref, dinv_ref, w2t_ref, o1_ref, g1_ref):
        dinv = dinv_ref[:, 0:1]
        atg2 = dinv * (t2_ref[0] + t2_ref[1] + g2_ref[...])
        go1 = atg2 @ w2t_ref[...]
        o1 = o1_ref[...]
        g1_ref[...] = dinv * (go1 * (1.0 - o1 * o1))

    return _tc(f, jax.ShapeDtypeStruct((np_rows, 2 * d), jnp.float32),
               t2, g2, dinv16, W2T, o1)


def _stage_update(t1, g1, dinv16, W1T, W1, Xc, Yc, np_rows, d):
    def f(t1_ref, g1_ref, dinv_ref, w1t_ref, w1_ref, x_ref, y_ref,
          xn_ref, yn_ref, m1_ref):
        dinv = dinv_ref[:, 0:1]
        g = (dinv * (t1_ref[0] + t1_ref[1] + g1_ref[...])) @ w1t_ref[...]
        xn = x_ref[...] + g[:, d:]
        yn = y_ref[...] - g[:, :d]
        xn_ref[...] = xn
        yn_ref[...] = yn
        m1_ref[...] = dinv * (xn @ w1_ref[0:d, :] + yn @ w1_ref[d:, :])

    return _tc(f, [jax.ShapeDtypeStruct((np_rows, d), jnp.float32),
                   jax.ShapeDtypeStruct((np_rows, d), jnp.float32),
                   jax.ShapeDtypeStruct((np_rows, 2 * d), jnp.float32)],
               t1, g1, dinv16, W1T, W1, Xc, Yc)


def _stage_dec(Xc, W_dec, b_dec, np_rows, nclass):
    def f(x_ref, wdec_ref, bdec_ref, out_ref):
        out_ref[...] = x_ref[...] @ wdec_ref[...] + bdec_ref[...]

    return _tc(f, jax.ShapeDtypeStruct((np_rows, nclass), jnp.float32),
               Xc, W_dec, b_dec)


def _pad_idx(idx, feat, fillvec):
    e = idx.shape[0]
    chunk, _, _ = _ring_for(feat)
    epc0, epc1 = _layout(e, feat)
    n0 = _NS * epc0
    total = _NS * (epc0 + epc1)
    npad = total - e
    fill = jnp.tile(fillvec, -(-npad // fillvec.shape[0]))[:npad]
    flat = jnp.concatenate([idx, fill])
    ch0, ch1 = epc0 // chunk, epc1 // chunk
    p0 = flat[:n0].reshape(_NS, ch0, chunk)
    p1 = flat[n0:].reshape(_NS, ch1, chunk)
    if ch0 > ch1:
        extra = jnp.tile(fillvec, -(-(_NS * (ch0 - ch1) * chunk)
                                    // fillvec.shape[0]))
        extra = extra[:_NS * (ch0 - ch1) * chunk].reshape(
            _NS, ch0 - ch1, chunk)
        p1 = jnp.concatenate([p1, extra], axis=1)
    return jnp.concatenate([p0, p1], axis=0)


def kernel(x, edge_index, W_enc, b_enc, W1, b1, W2, b2, W3, b3, W_dec, b_dec):
    n = x.shape[0]
    e = edge_index.shape[1]
    d = W_enc.shape[1]
    nclass = W_dec.shape[1]
    nlayers = 2

    np_rows = -(-(n + 128) // _NS) * _NS

    src = edge_index[0].astype(jnp.int32)
    dst = edge_index[1].astype(jnp.int32)
    cyc = jnp.arange(128, dtype=jnp.int32)
    gfill = n + cyc
    sfill = (cyc * 79) % n
    sg64, ss64 = _pad_idx(src, d, gfill), _pad_idx(src, d, sfill)
    dg64, ds64 = _pad_idx(dst, d, gfill), _pad_idx(dst, d, sfill)
    sg128, ss128 = _pad_idx(src, 2 * d, gfill), _pad_idx(src, 2 * d, sfill)
    dg128, ds128 = _pad_idx(dst, 2 * d, gfill), _pad_idx(dst, 2 * d, sfill)
    zidx = _pad_idx(jnp.arange(e, dtype=jnp.int32) % 2048, 16, 2048 + cyc)
    ones_tab = jnp.concatenate([jnp.ones((2048, 16), jnp.float32),
                                jnp.zeros((128, 16), jnp.float32)])

    x_p = jnp.concatenate(
        [x, jnp.zeros((np_rows - n, x.shape[1]), jnp.float32)])
    b_enc_r = b_enc.reshape(1, d)
    b1_r = b1.reshape(1, 2 * d)
    b2_r = b2.reshape(1, d)
    w3row = W3.reshape(1, d)
    b_dec_r = b_dec.reshape(1, nclass)
    W1T = W1.T
    W2T = W2.T

    sc16 = _make_spmm(np_rows, 16, *_layout(e, 16))
    sc64 = _make_spmm(np_rows, d, *_layout(e, d))
    sc128 = _make_spmm(np_rows, 2 * d, *_layout(e, 2 * d))

    degraw = sc16(ones_tab, zidx, ds64)
    dinv16, Y, M1 = _stage_enc(x_p, degraw, W_enc, b_enc_r, W1, np_rows, d, n)
    craw = sc16(dinv16, dg64, ss64)

    X = Y
    for layer in range(nlayers):
        S1 = sc128(M1, sg128, ds128)
        o1, M2 = _stage_fwd1(S1, M1, dinv16, b1_r, W2, np_rows, d)
        S2 = sc64(M2, sg64, ds64)
        G2 = _stage_fwd2(S2, M2, dinv16, craw, b2_r, w3row, np_rows, d)
        T2 = sc64(G2, dg64, ss64)
        G1 = _stage_bwd1(T2, G2, dinv16, W2T, o1, np_rows, d)
        T1 = sc128(G1, dg128, ss128)
        X, Y, M1 = _stage_update(T1, G1, dinv16, W1T, W1, X, Y, np_rows, d)

    out = _stage_dec(X, W_dec, b_dec_r, np_rows, nclass)
    return out[:n]


# speedup vs baseline: 2.4792x; 1.0069x over previous
import functools

import jax
import jax.numpy as jnp
from jax import lax
from jax.experimental import pallas as pl
from jax.experimental.pallas import tpu as pltpu
from jax.experimental.pallas import tpu_sc as plsc

_NC = 2
_NS = 16
_NW = _NC * _NS
_CHUNK = 128


def _ring_for(feat):
    return (64, 3, 2) if feat >= 128 else (_CHUNK, 4, 1)


def _share_for(feat):
    return 0.5


def _layout(e, feat):
    chunk, nbuf, phases = _ring_for(feat)
    q = chunk * nbuf * phases
    per0 = int(_share_for(feat) * e / _NS)
    epc0 = -(-per0 // q) * q
    rem = max(e - _NS * epc0, 0)
    per1 = -(-rem // _NS)
    epc1 = max(-(-per1 // q) * q, q)
    return epc0, epc1


def _make_spmm(np_rows, feat, epc0, epc1):
    rows_per_tile = np_rows // _NS
    chunk, nbuf, phases = _ring_for(feat)
    ch0 = epc0 // chunk
    ng0 = ch0 // nbuf
    ng1 = (epc1 // chunk) // nbuf
    chp = ch0 // phases
    gpp = chp // nbuf
    mesh = plsc.VectorSubcoreMesh(
        core_axis_name="c", subcore_axis_name="s",
        num_cores=_NC, num_subcores=_NS)

    @functools.partial(
        pl.kernel,
        out_type=jax.ShapeDtypeStruct((_NC, np_rows, feat), jnp.float32),
        mesh=mesh,
        scratch_types=(
            [pltpu.VMEM((chp, chunk), jnp.int32),
             pltpu.VMEM((chp, chunk), jnp.int32)]
            + [pltpu.VMEM((chunk, feat), jnp.float32)] * nbuf
            + [pltpu.VMEM_SHARED((np_rows, feat), jnp.float32)]
            + [pltpu.SemaphoreType.DMA] * (2 * nbuf)
        ),
        compiler_params=pltpu.CompilerParams(use_tc_tiling_on_sc=False),
    )
    def spmm(table, gidx, sidx, out, gidx_v, sidx_v, *rest):
        bufs = rest[:nbuf]
        acc = rest[nbuf]
        gsems = rest[nbuf + 1:2 * nbuf + 1]
        ssems = rest[2 * nbuf + 1:]
        c = lax.axis_index("c")
        s = lax.axis_index("s")
        wid = c * _NS + s
        r0 = s * rows_per_tile
        ng_c = jnp.where(c == 0, ng0, ng1)
        with jax.named_scope("zero"):
            zv = jnp.zeros((16,), jnp.float32)

            def zfill(i, carry):
                for kk in range(feat // 16):
                    bufs[0][i, pl.ds(kk * 16, 16)] = zv
                return carry

            lax.fori_loop(0, chunk, zfill, 0)
            nfull = rows_per_tile // chunk
            rem = rows_per_tile % chunk
            for k in range(nfull):
                pltpu.sync_copy(bufs[0],
                                acc.at[pl.ds(r0 + k * chunk, chunk)])
            if rem:
                pltpu.sync_copy(bufs[0].at[pl.ds(0, rem)],
                                acc.at[pl.ds(r0 + nfull * chunk, rem)])
            plsc.subcore_barrier()

        for ph in range(phases):
            ngp = jnp.clip(ng_c - ph * gpp, 0, gpp)

            @pl.when(ngp > 0)
            def _(ph=ph, ngp=ngp):
                with jax.named_scope("stage"):
                    pltpu.sync_copy(gidx.at[wid, pl.ds(ph * chp, chp)],
                                    gidx_v)
                    pltpu.sync_copy(sidx.at[wid, pl.ds(ph * chp, chp)],
                                    sidx_v)
                for b in range(nbuf):
                    pltpu.async_copy(table.at[gidx_v.at[b]], bufs[b],
                                     gsems[b])

                def body(i, carry):
                    j0 = i * nbuf
                    for b in range(nbuf):
                        pltpu.make_async_copy(table.at[gidx_v.at[j0 + b]],
                                              bufs[b], gsems[b]).wait()
                        pltpu.async_copy(bufs[b], acc.at[sidx_v.at[j0 + b]],
                                         ssems[b], add=True)
                    for b in range(nbuf):
                        @pl.when(i + 1 < ngp)
                        def _(b=b, j0=j0):
                            pltpu.make_async_copy(
                                bufs[b], acc.at[sidx_v.at[j0 + b]],
                                ssems[b]).wait()
                            pltpu.async_copy(
                                table.at[gidx_v.at[j0 + nbuf + b]],
                                bufs[b], gsems[b])
                    return carry

                with jax.named_scope("ring"):
                    lax.fori_loop(0, ngp, body, 0)
                    for b in range(nbuf):
                        jlast = (ngp - 1) * nbuf + b
                        pltpu.make_async_copy(bufs[b],
                                              acc.at[sidx_v.at[jlast]],
                                              ssems[b]).wait()

        with jax.named_scope("cpout"):
            plsc.subcore_barrier()
            pltpu.sync_copy(acc.at[pl.ds(r0, rows_per_tile)],
                            out.at[c, pl.ds(r0, rows_per_tile)])

    return spmm


def _tc(fn, out_shapes, *args):
    return pl.pallas_call(fn, out_shape=out_shapes)(*args)


def _stage_y(x, W_enc, b_enc, np_rows, d):
    def f(x_ref, wenc_ref, benc_ref, y_ref):
        y_ref[...] = jnp.maximum(x_ref[...] @ wenc_ref[...] + benc_ref[...],
                                 0.0)

    return _tc(f, jax.ShapeDtypeStruct((np_rows, d), jnp.float32),
               x, W_enc, b_enc)


def _stage_enc(Y, degraw, W1, np_rows, d, n):
    def f(y_ref, deg_ref, w1_ref, dinv16_ref, m1_ref):
        deg = deg_ref[0, :, 0:1] + deg_ref[1, :, 0:1] + 1.0
        row = lax.broadcasted_iota(jnp.int32, (np_rows, 1), 0)
        dinv = jnp.where(row < n, lax.rsqrt(deg), 0.0)
        dinv16_ref[...] = jnp.broadcast_to(dinv, dinv16_ref.shape)
        wsum = w1_ref[0:d, :] + w1_ref[d:, :]
        m1_ref[...] = dinv * (y_ref[...] @ wsum)

    return _tc(f, [jax.ShapeDtypeStruct((np_rows, 16), jnp.float32),
                   jax.ShapeDtypeStruct((np_rows, 2 * d), jnp.float32)],
               Y, degraw, W1)


def _stage_fwd1(s1, m1, dinv16, b1, W2, np_rows, d):
    def f(s1_ref, m1_ref, dinv_ref, b1_ref, w2_ref, o1_ref, m2_ref):
        dinv = dinv_ref[:, 0:1]
        z1 = dinv * (s1_ref[0] + s1_ref[1] + m1_ref[...]) + b1_ref[...]
        o1 = jnp.tanh(z1)
        o1_ref[...] = o1
        m2_ref[...] = dinv * (o1 @ w2_ref[...])

    return _tc(f, [jax.ShapeDtypeStruct((np_rows, 2 * d), jnp.float32),
                   jax.ShapeDtypeStruct((np_rows, d), jnp.float32)],
               s1, m1, dinv16, b1, W2)


def _stage_fwd2(s2, m2, dinv16, craw, b2, w3row, np_rows, d):
    def f(s2_ref, m2_ref, dinv_ref, craw_ref, b2_ref, w3_ref, g2_ref):
        dinv = dinv_ref[:, 0:1]
        z2 = dinv * (s2_ref[0] + s2_ref[1] + m2_ref[...]) + b2_ref[...]
        o2 = jnp.tanh(z2)
        c = dinv * (craw_ref[0, :, 0:1] + craw_ref[1, :, 0:1]) + dinv * dinv
        gz2 = (c * w3_ref[...]) * (1.0 - o2 * o2)
        g2_ref[...] = dinv * gz2

    return _tc(f, jax.ShapeDtypeStruct((np_rows, d), jnp.float32),
               s2, m2, dinv16, craw, b2, w3row)


def _stage_bwd1(t2, g2, dinv16, W2T, o1, np_rows, d):
    def f(t2_ref, g2_ref, dinv_ref, w2t_ref, o1_ref, g1_ref):
        dinv = dinv_ref[:, 0:1]
        atg2 = dinv * (t2_ref[0] + t2_ref[1] + g2_ref[...])
        go1 = atg2 @ w2t_ref[...]
        o1 = o1_ref[...]
        g1_ref[...] = dinv * (go1 * (1.0 - o1 * o1))

    return _tc(f, jax.ShapeDtypeStruct((np_rows, 2 * d), jnp.float32),
               t2, g2, dinv16, W2T, o1)


def _stage_update(t1, g1, dinv16, W1T, W1, Xc, Yc, np_rows, d):
    def f(t1_ref, g1_ref, dinv_ref, w1t_ref, w1_ref, x_ref, y_ref,
          xn_ref, yn_ref, m1_ref):
        dinv = dinv_ref[:, 0:1]
        g = (dinv * (t1_ref[0] + t1_ref[1] + g1_ref[...])) @ w1t_ref[...]
        xn = x_ref[...] + g[:, d:]
        yn = y_ref[...] - g[:, :d]
        xn_ref[...] = xn
        yn_ref[...] = yn
        m1_ref[...] = dinv * (xn @ w1_ref[0:d, :] + yn @ w1_ref[d:, :])

    return _tc(f, [jax.ShapeDtypeStruct((np_rows, d), jnp.float32),
                   jax.ShapeDtypeStruct((np_rows, d), jnp.float32),
                   jax.ShapeDtypeStruct((np_rows, 2 * d), jnp.float32)],
               t1, g1, dinv16, W1T, W1, Xc, Yc)


def _stage_update_dec(t1, g1, dinv16, W1T, Xc, W_dec, b_dec, np_rows, d,
                      nclass):
    def f(t1_ref, g1_ref, dinv_ref, w1t_ref, x_ref, wdec_ref, bdec_ref,
          out_ref):
        dinv = dinv_ref[:, 0:1]
        g = (dinv * (t1_ref[0] + t1_ref[1] + g1_ref[...])) @ w1t_ref[...]
        xn = x_ref[...] + g[:, d:]
        out_ref[...] = xn @ wdec_ref[...] + bdec_ref[...]

    return _tc(f, jax.ShapeDtypeStruct((np_rows, nclass), jnp.float32),
               t1, g1, dinv16, W1T, Xc, W_dec, b_dec)


def _pad_idx(idx, feat, fillvec):
    e = idx.shape[0]
    chunk, _, _ = _ring_for(feat)
    epc0, epc1 = _layout(e, feat)
    n0 = _NS * epc0
    total = _NS * (epc0 + epc1)
    npad = total - e
    fill = jnp.tile(fillvec, -(-npad // fillvec.shape[0]))[:npad]
    flat = jnp.concatenate([idx, fill])
    ch0, ch1 = epc0 // chunk, epc1 // chunk
    p0 = flat[:n0].reshape(_NS, ch0, chunk)
    p1 = flat[n0:].reshape(_NS, ch1, chunk)
    if ch0 > ch1:
        extra = jnp.tile(fillvec, -(-(_NS * (ch0 - ch1) * chunk)
                                    // fillvec.shape[0]))
        extra = extra[:_NS * (ch0 - ch1) * chunk].reshape(
            _NS, ch0 - ch1, chunk)
        p1 = jnp.concatenate([p1, extra], axis=1)
    return jnp.concatenate([p0, p1], axis=0)


def kernel(x, edge_index, W_enc, b_enc, W1, b1, W2, b2, W3, b3, W_dec, b_dec):
    n = x.shape[0]
    e = edge_index.shape[1]
    d = W_enc.shape[1]
    nclass = W_dec.shape[1]
    nlayers = 2

    np_rows = -(-(n + 128) // _NS) * _NS

    src = edge_index[0].astype(jnp.int32)
    dst = edge_index[1].astype(jnp.int32)
    cyc = jnp.arange(128, dtype=jnp.int32)
    gfill = n + cyc
    sfill = (cyc * 79) % n
    sg64, ss64 = _pad_idx(src, d, gfill), _pad_idx(src, d, sfill)
    dg64, ds64 = _pad_idx(dst, d, gfill), _pad_idx(dst, d, sfill)
    sg128, ss128 = _pad_idx(src, 2 * d, gfill), _pad_idx(src, 2 * d, sfill)
    dg128, ds128 = _pad_idx(dst, 2 * d, gfill), _pad_idx(dst, 2 * d, sfill)
    zidx = _pad_idx(jnp.arange(e, dtype=jnp.int32) % 2048, 16, 2048 + cyc)
    ones_tab = jnp.concatenate([jnp.ones((2048, 16), jnp.float32),
                                jnp.zeros((128, 16), jnp.float32)])

    x_p = jnp.concatenate(
        [x, jnp.zeros((np_rows - n, x.shape[1]), jnp.float32)])
    b_enc_r = b_enc.reshape(1, d)
    b1_r = b1.reshape(1, 2 * d)
    b2_r = b2.reshape(1, d)
    w3row = W3.reshape(1, d)
    b_dec_r = b_dec.reshape(1, nclass)
    W1T = W1.T
    W2T = W2.T

    sc16 = _make_spmm(np_rows, 16, *_layout(e, 16))
    sc64 = _make_spmm(np_rows, d, *_layout(e, d))
    sc128 = _make_spmm(np_rows, 2 * d, *_layout(e, 2 * d))

    degraw = sc16(ones_tab, zidx, ds64)
    Y = _stage_y(x_p, W_enc, b_enc_r, np_rows, d)
    dinv16, M1 = _stage_enc(Y, degraw, W1, np_rows, d, n)
    craw = sc16(dinv16, dg64, ss64)

    X = Y
    for layer in range(nlayers):
        S1 = sc128(M1, sg128, ds128)
        o1, M2 = _stage_fwd1(S1, M1, dinv16, b1_r, W2, np_rows, d)
        S2 = sc64(M2, sg64, ds64)
        G2 = _stage_fwd2(S2, M2, dinv16, craw, b2_r, w3row, np_rows, d)
        T2 = sc64(G2, dg64, ss64)
        G1 = _stage_bwd1(T2, G2, dinv16, W2T, o1, np_rows, d)
        T1 = sc128(G1, dg128, ss128)
        if layer + 1 < nlayers:
            X, Y, M1 = _stage_update(T1, G1, dinv16, W1T, W1, X, Y,
                                     np_rows, d)
        else:
            out = _stage_update_dec(T1, G1, dinv16, W1T, X, W_dec,
                                    b_dec_r, np_rows, d, nclass)

    return out[:n]
